# baseline (speedup 1.0000x reference)
import jax
import jax.numpy as jnp
import numpy as np
from jax.experimental import pallas as pl

N_PART = 13
NP = 16
DIM = 3
HID = 32
LW = NP * HID
N_LAYERS = 4
SIGMA_DATA = 0.68
BATCH = 4096
BB = 64


def _silu(x):
    return x * (0.5 * jnp.tanh(0.5 * x) + 0.5)


def _fused_kernel(xt_ref, t_ref, wsin_ref, wcos_ref, embb_ref,
                  wat_ref, wb_ref, wr_ref, we_ref, b1t_ref,
                  bdw2_ref, b2t_ref, bdc1_ref, c1bt_ref, c2s_ref, summ_ref,
                  wn1h_ref, wn1a_ref, bn1_ref, wn2_ref, bn2_ref,
                  out_ref):
    f32 = jnp.float32
    bf16 = jnp.bfloat16
    R = BB * NP
    xt = xt_ref[...]
    t = t_ref[...]

    c_in = jax.lax.rsqrt(t * t + SIGMA_DATA ** 2)
    x = xt * c_in[:, :, None]

    k = jax.lax.broadcasted_iota(jnp.int32, (1, HID), 1).astype(f32)
    freqs = jnp.exp((-np.log(10000.0) / 31.0) * k)
    args = (jnp.log(t) * 0.25) * freqs
    h0 = (jnp.dot(jnp.sin(args), wsin_ref[...], preferred_element_type=f32)
          + jnp.dot(jnp.cos(args), wcos_ref[...], preferred_element_type=f32)
          + embb_ref[...])
    h = jnp.broadcast_to(h0[:, None, :], (BB, NP, HID)).reshape(R, HID)

    crd = [x[:, :, d:d + 1].reshape(R, 1) for d in range(DIM)]

    i_id = jax.lax.broadcasted_iota(jnp.int32, (R, NP), 0) % NP
    j_id = jax.lax.broadcasted_iota(jnp.int32, (R, NP), 1)
    mask_j = (j_id < N_PART) & (j_id != i_id)
    i_idw = jax.lax.broadcasted_iota(jnp.int32, (R, LW), 0) % NP
    j_idw = jax.lax.broadcasted_iota(jnp.int32, (R, LW), 1) // HID
    mask_w = (j_idw < N_PART) & (j_idw != i_idw)

    def pair_geom(c):
        diffs = []
        radial = None
        for d in range(DIM):
            cl = jnp.swapaxes(c[d].reshape(BB, NP, 1), 1, 2)
            cj = jnp.broadcast_to(cl, (BB, NP, NP)).reshape(R, NP)
            dd = jnp.broadcast_to(c[d], (R, NP)) - cj
            diffs.append(dd)
            radial = dd * dd if radial is None else radial + dd * dd
        return diffs, radial

    d0, ea_j = pair_geom(crd)

    for l in range(N_LAYERS):
        if l == 0:
            diffs, radial_j = d0, ea_j
        else:
            diffs, radial_j = pair_geom(crd)
        inv = 1.0 / (jnp.sqrt(radial_j + 1e-8) + 1.0)

        hb = h.astype(bf16)
        hwa_t = jnp.dot(hb, wat_ref[l], preferred_element_type=f32)
        hwb = jnp.dot(hb, wb_ref[l], preferred_element_type=f32)
        hwb3 = hwb.reshape(BB, NP, HID)
        hwb_pk = jnp.concatenate([hwb3[:, j, :] for j in range(NP)],
                                 axis=1)
        hwb_b = jnp.broadcast_to(hwb_pk[:, None, :],
                                 (BB, NP, LW)).reshape(R, LW)
        e_lin = (hwa_t + hwb_b
                 + jnp.dot(radial_j.astype(bf16), wr_ref[l],
                           preferred_element_type=f32)
                 + jnp.dot(ea_j.astype(bf16), we_ref[l],
                           preferred_element_type=f32)
                 + b1t_ref[l])
        m1 = _silu(e_lin)
        m = _silu(jnp.dot(m1.astype(bf16), bdw2_ref[l],
                          preferred_element_type=f32) + b2t_ref[l])
        s1 = _silu(jnp.dot(m.astype(bf16), bdc1_ref[l],
                           preferred_element_type=f32) + c1bt_ref[l])
        scal_j = jnp.dot(s1.astype(bf16), c2s_ref[l],
                         preferred_element_type=f32)
        scal_m = jnp.where(mask_j, scal_j, 0.0)
        for d in range(DIM):
            upd = jnp.sum(diffs[d] * inv * scal_m, axis=1, keepdims=True)
            crd[d] = crd[d] + upd

        if l < N_LAYERS - 1:
            m_masked = jnp.where(mask_w, m, 0.0)
            agg = jnp.dot(m_masked.astype(bf16), summ_ref[...],
                          preferred_element_type=f32)
            n1 = _silu(jnp.dot(hb, wn1h_ref[l], preferred_element_type=f32)
                       + jnp.dot(agg.astype(bf16), wn1a_ref[l],
                                 preferred_element_type=f32)
                       + bn1_ref[l])
            h = h + jnp.dot(n1.astype(bf16), wn2_ref[l],
                            preferred_element_type=f32) + bn2_ref[l]

    nmask = jax.lax.broadcasted_iota(jnp.int32, (1, NP, 1), 1) < N_PART
    xp = jnp.concatenate(crd, axis=1).reshape(BB, NP, DIM)
    vec = xp - x
    vec = vec - jnp.sum(jnp.where(nmask, vec, 0.0), axis=1,
                        keepdims=True) * (1.0 / N_PART)
    c_skip = (SIGMA_DATA ** 2) * (c_in * c_in)
    c_out = t * SIGMA_DATA * c_in
    x0 = c_skip[:, :, None] * xt + c_out[:, :, None] * vec
    x0 = x0 - jnp.sum(jnp.where(nmask, x0, 0.0), axis=1,
                      keepdims=True) * (1.0 / N_PART)
    out_ref[...] = x0


@jax.jit
def kernel(xt, t, params):
    B = xt.shape[0]
    xt_p = jnp.pad(xt, ((0, 0), (0, NP - N_PART), (0, 0)))
    t2 = t[:, None]

    L = params["layers"]
    eye = jnp.eye(NP, dtype=jnp.float32)
    stk = lambda f: jnp.stack([f(lp) for lp in L])
    bcast = jnp.bfloat16
    wat = stk(lambda lp: jnp.tile(lp["edge_w1"][:HID],
                                  (1, NP))).astype(bcast)
    wb = stk(lambda lp: lp["edge_w1"][HID:2 * HID]).astype(bcast)
    wr = stk(lambda lp: jnp.kron(eye, lp["edge_w1"][2 * HID:2 * HID + 1])
             ).astype(bcast)
    we = stk(lambda lp: jnp.kron(eye, lp["edge_w1"][2 * HID + 1:2 * HID + 2])
             ).astype(bcast)
    b1t = stk(lambda lp: jnp.tile(lp["edge_b1"][None], (1, NP)))
    bdw2 = stk(lambda lp: jnp.kron(eye, lp["edge_w2"])).astype(jnp.bfloat16)
    b2t = stk(lambda lp: jnp.tile(lp["edge_b2"][None], (1, NP)))
    bdc1 = stk(lambda lp: jnp.kron(eye, lp["coord_w1"])).astype(jnp.bfloat16)
    c1bt = stk(lambda lp: jnp.tile(lp["coord_b1"][None], (1, NP)))
    c2s = stk(lambda lp: jnp.kron(eye, lp["coord_w2"])).astype(jnp.bfloat16)
    summ = jnp.tile(jnp.eye(HID, dtype=jnp.float32),
                    (NP, 1)).astype(jnp.bfloat16)
    wn1h = stk(lambda lp: lp["node_w1"][:HID]).astype(bcast)
    wn1a = stk(lambda lp: lp["node_w1"][HID:]).astype(bcast)
    bn1 = stk(lambda lp: lp["node_b1"][None])
    wn2 = stk(lambda lp: lp["node_w2"]).astype(bcast)
    bn2 = stk(lambda lp: lp["node_b2"][None])
    wsin = params["emb_w"][:HID]
    wcos = params["emb_w"][HID:]
    embb = params["emb_b"][None]

    grid = B // BB
    full = lambda s: pl.BlockSpec(s, lambda b: (0,) * len(s))
    out = pl.pallas_call(
        _fused_kernel,
        grid=(grid,),
        in_specs=[
            pl.BlockSpec((BB, NP, DIM), lambda b: (b, 0, 0)),
            pl.BlockSpec((BB, 1), lambda b: (b, 0)),
            full((HID, HID)), full((HID, HID)), full((1, HID)),
            full((N_LAYERS, HID, LW)), full((N_LAYERS, HID, HID)),
            full((N_LAYERS, NP, LW)), full((N_LAYERS, NP, LW)),
            full((N_LAYERS, 1, LW)),
            full((N_LAYERS, LW, LW)), full((N_LAYERS, 1, LW)),
            full((N_LAYERS, LW, LW)), full((N_LAYERS, 1, LW)),
            full((N_LAYERS, LW, NP)), full((LW, HID)),
            full((N_LAYERS, HID, HID)), full((N_LAYERS, HID, HID)),
            full((N_LAYERS, 1, HID)),
            full((N_LAYERS, HID, HID)), full((N_LAYERS, 1, HID)),
        ],
        out_specs=pl.BlockSpec((BB, NP, DIM), lambda b: (b, 0, 0)),
        out_shape=jax.ShapeDtypeStruct((B, NP, DIM), jnp.float32),
    )(xt_p, t2, wsin, wcos, embb, wat, wb, wr, we, b1t,
      bdw2, b2t, bdc1, c1bt, c2s, summ, wn1h, wn1a, bn1, wn2, bn2)
    return out[:, :N_PART, :]


# speedup vs baseline: 1.0053x; 1.0053x over previous
import jax
import jax.numpy as jnp
import numpy as np
from jax.experimental import pallas as pl

N_PART = 13
NP = 16
DIM = 3
HID = 32
LW = NP * HID
N_LAYERS = 4
SIGMA_DATA = 0.68
BATCH = 4096
BB = 64


def _silu(x):
    return x * (0.5 * jnp.tanh(0.5 * x) + 0.5)


def _fused_kernel(xt_ref, t_ref, wsin_ref, wcos_ref, embb_ref,
                  wat_ref, wb_ref, wr_ref, we_ref, b1t_ref,
                  bdw2_ref, b2t_ref, bdc1_ref, c1bt_ref, c2s_ref, summ_ref,
                  wn1h_ref, wn1a_ref, bn1_ref, wn2_ref, bn2_ref,
                  out_ref):
    f32 = jnp.float32
    bf16 = jnp.bfloat16
    R = BB * NP
    xt = xt_ref[...]
    t = t_ref[...]

    c_in = jax.lax.rsqrt(t * t + SIGMA_DATA ** 2)
    x = xt * c_in[:, :, None]

    k = jax.lax.broadcasted_iota(jnp.int32, (1, HID), 1).astype(f32)
    freqs = jnp.exp((-np.log(10000.0) / 31.0) * k)
    args = (jnp.log(t) * 0.25) * freqs
    h0 = (jnp.dot(jnp.sin(args), wsin_ref[...], preferred_element_type=f32)
          + jnp.dot(jnp.cos(args), wcos_ref[...], preferred_element_type=f32)
          + embb_ref[...])
    h = jnp.broadcast_to(h0[:, None, :], (BB, NP, HID)).reshape(R, HID)

    crd = [x[:, :, d:d + 1].reshape(R, 1) for d in range(DIM)]

    i_id = jax.lax.broadcasted_iota(jnp.int32, (R, NP), 0) % NP
    j_id = jax.lax.broadcasted_iota(jnp.int32, (R, NP), 1)
    mask_j = (j_id < N_PART) & (j_id != i_id)
    i_idw = jax.lax.broadcasted_iota(jnp.int32, (R, LW), 0) % NP
    j_idw = jax.lax.broadcasted_iota(jnp.int32, (R, LW), 1) // HID
    mask_w = (j_idw < N_PART) & (j_idw != i_idw)

    def pair_geom(c):
        diffs = []
        radial = None
        for d in range(DIM):
            cl = jnp.swapaxes(c[d].reshape(BB, NP, 1), 1, 2)
            cj = jnp.broadcast_to(cl, (BB, NP, NP)).reshape(R, NP)
            dd = jnp.broadcast_to(c[d], (R, NP)) - cj
            diffs.append(dd)
            radial = dd * dd if radial is None else radial + dd * dd
        return diffs, radial

    d0, ea_j = pair_geom(crd)

    for l in range(N_LAYERS):
        if l == 0:
            diffs, radial_j = d0, ea_j
        else:
            diffs, radial_j = pair_geom(crd)
        inv = 1.0 / (jnp.sqrt(radial_j + 1e-8) + 1.0)

        hb = h.astype(bf16)
        hwa_t = jnp.dot(hb, wat_ref[l], preferred_element_type=f32)
        hwb = jnp.dot(hb, wb_ref[l], preferred_element_type=f32)
        hwb3 = hwb.reshape(BB, NP, HID)
        hwb_pk = jnp.concatenate([hwb3[:, j, :] for j in range(NP)],
                                 axis=1) + b1t_ref[l]
        hwb_b = jnp.broadcast_to(hwb_pk[:, None, :],
                                 (BB, NP, LW)).reshape(R, LW)
        e_lin = (hwa_t + hwb_b
                 + jnp.dot(radial_j.astype(bf16), wr_ref[l],
                           preferred_element_type=f32)
                 + jnp.dot(ea_j.astype(bf16), we_ref[l],
                           preferred_element_type=f32))
        m1 = _silu(e_lin)
        m = _silu(jnp.dot(m1.astype(bf16), bdw2_ref[l],
                          preferred_element_type=f32) + b2t_ref[l])
        s1 = _silu(jnp.dot(m.astype(bf16), bdc1_ref[l],
                           preferred_element_type=f32) + c1bt_ref[l])
        scal_j = jnp.dot(s1.astype(bf16), c2s_ref[l],
                         preferred_element_type=f32)
        scal_m = jnp.where(mask_j, scal_j, 0.0)
        for d in range(DIM):
            upd = jnp.sum(diffs[d] * inv * scal_m, axis=1, keepdims=True)
            crd[d] = crd[d] + upd

        if l < N_LAYERS - 1:
            m_masked = jnp.where(mask_w, m, 0.0)
            agg = jnp.dot(m_masked.astype(bf16), summ_ref[...],
                          preferred_element_type=f32)
            n1 = _silu(jnp.dot(hb, wn1h_ref[l], preferred_element_type=f32)
                       + jnp.dot(agg.astype(bf16), wn1a_ref[l],
                                 preferred_element_type=f32)
                       + bn1_ref[l])
            h = h + jnp.dot(n1.astype(bf16), wn2_ref[l],
                            preferred_element_type=f32) + bn2_ref[l]

    nmask = jax.lax.broadcasted_iota(jnp.int32, (1, NP, 1), 1) < N_PART
    xp = jnp.concatenate(crd, axis=1).reshape(BB, NP, DIM)
    vec = xp - x
    vec = vec - jnp.sum(jnp.where(nmask, vec, 0.0), axis=1,
                        keepdims=True) * (1.0 / N_PART)
    c_skip = (SIGMA_DATA ** 2) * (c_in * c_in)
    c_out = t * SIGMA_DATA * c_in
    x0 = c_skip[:, :, None] * xt + c_out[:, :, None] * vec
    x0 = x0 - jnp.sum(jnp.where(nmask, x0, 0.0), axis=1,
                      keepdims=True) * (1.0 / N_PART)
    out_ref[...] = x0


@jax.jit
def kernel(xt, t, params):
    B = xt.shape[0]
    xt_p = jnp.pad(xt, ((0, 0), (0, NP - N_PART), (0, 0)))
    t2 = t[:, None]

    L = params["layers"]
    eye = jnp.eye(NP, dtype=jnp.float32)
    stk = lambda f: jnp.stack([f(lp) for lp in L])
    bcast = jnp.bfloat16
    wat = stk(lambda lp: jnp.tile(lp["edge_w1"][:HID],
                                  (1, NP))).astype(bcast)
    wb = stk(lambda lp: lp["edge_w1"][HID:2 * HID]).astype(bcast)
    wr = stk(lambda lp: jnp.kron(eye, lp["edge_w1"][2 * HID:2 * HID + 1])
             ).astype(bcast)
    we = stk(lambda lp: jnp.kron(eye, lp["edge_w1"][2 * HID + 1:2 * HID + 2])
             ).astype(bcast)
    b1t = stk(lambda lp: jnp.tile(lp["edge_b1"][None], (1, NP)))
    bdw2 = stk(lambda lp: jnp.kron(eye, lp["edge_w2"])).astype(jnp.bfloat16)
    b2t = stk(lambda lp: jnp.tile(lp["edge_b2"][None], (1, NP)))
    bdc1 = stk(lambda lp: jnp.kron(eye, lp["coord_w1"])).astype(jnp.bfloat16)
    c1bt = stk(lambda lp: jnp.tile(lp["coord_b1"][None], (1, NP)))
    c2s = stk(lambda lp: jnp.kron(eye, lp["coord_w2"])).astype(jnp.bfloat16)
    summ = jnp.tile(jnp.eye(HID, dtype=jnp.float32),
                    (NP, 1)).astype(jnp.bfloat16)
    wn1h = stk(lambda lp: lp["node_w1"][:HID]).astype(bcast)
    wn1a = stk(lambda lp: lp["node_w1"][HID:]).astype(bcast)
    bn1 = stk(lambda lp: lp["node_b1"][None])
    wn2 = stk(lambda lp: lp["node_w2"]).astype(bcast)
    bn2 = stk(lambda lp: lp["node_b2"][None])
    wsin = params["emb_w"][:HID]
    wcos = params["emb_w"][HID:]
    embb = params["emb_b"][None]

    grid = B // BB
    full = lambda s: pl.BlockSpec(s, lambda b: (0,) * len(s))
    out = pl.pallas_call(
        _fused_kernel,
        grid=(grid,),
        in_specs=[
            pl.BlockSpec((BB, NP, DIM), lambda b: (b, 0, 0)),
            pl.BlockSpec((BB, 1), lambda b: (b, 0)),
            full((HID, HID)), full((HID, HID)), full((1, HID)),
            full((N_LAYERS, HID, LW)), full((N_LAYERS, HID, HID)),
            full((N_LAYERS, NP, LW)), full((N_LAYERS, NP, LW)),
            full((N_LAYERS, 1, LW)),
            full((N_LAYERS, LW, LW)), full((N_LAYERS, 1, LW)),
            full((N_LAYERS, LW, LW)), full((N_LAYERS, 1, LW)),
            full((N_LAYERS, LW, NP)), full((LW, HID)),
            full((N_LAYERS, HID, HID)), full((N_LAYERS, HID, HID)),
            full((N_LAYERS, 1, HID)),
            full((N_LAYERS, HID, HID)), full((N_LAYERS, 1, HID)),
        ],
        out_specs=pl.BlockSpec((BB, NP, DIM), lambda b: (b, 0, 0)),
        out_shape=jax.ShapeDtypeStruct((B, NP, DIM), jnp.float32),
    )(xt_p, t2, wsin, wcos, embb, wat, wb, wr, we, b1t,
      bdw2, b2t, bdc1, c1bt, c2s, summ, wn1h, wn1a, bn1, wn2, bn2)
    return out[:, :N_PART, :]


# speedup vs baseline: 1.0059x; 1.0005x over previous
import jax
import jax.numpy as jnp
import numpy as np
from jax.experimental import pallas as pl

N_PART = 13
NP = 16
DIM = 3
HID = 32
LW = NP * HID
N_LAYERS = 4
SIGMA_DATA = 0.68
BATCH = 4096
BB = 64


def _silu(x):
    return x * (0.5 * jnp.tanh(0.5 * x) + 0.5)


def _fused_kernel(xt_ref, t_ref, wsin_ref, wcos_ref, embb_ref,
                  wat_ref, wb_ref, wr_ref, we_ref, b1t_ref,
                  bdw2_ref, b2t_ref, bdc1_ref, c1bt_ref, c2s_ref, summ_ref,
                  wn1h_ref, wn1a_ref, bn1_ref, wn2_ref, bn2_ref,
                  out_ref):
    f32 = jnp.float32
    bf16 = jnp.bfloat16
    R = BB * NP
    xt = xt_ref[...]
    t = t_ref[...]

    c_in = jax.lax.rsqrt(t * t + SIGMA_DATA ** 2)
    x = xt * c_in[:, :, None]

    k = jax.lax.broadcasted_iota(jnp.int32, (1, HID), 1).astype(f32)
    freqs = jnp.exp((-np.log(10000.0) / 31.0) * k)
    args = (jnp.log(t) * 0.25) * freqs
    h0 = (jnp.dot(jnp.sin(args), wsin_ref[...], preferred_element_type=f32)
          + jnp.dot(jnp.cos(args), wcos_ref[...], preferred_element_type=f32)
          + embb_ref[...])
    h = jnp.broadcast_to(h0[:, None, :], (BB, NP, HID)).reshape(R, HID)

    crd = [x[:, :, d:d + 1].reshape(R, 1) for d in range(DIM)]

    i_id = jax.lax.broadcasted_iota(jnp.int32, (R, NP), 0) % NP
    j_id = jax.lax.broadcasted_iota(jnp.int32, (R, NP), 1)
    mask_j = (j_id < N_PART) & (j_id != i_id)
    i_idw = jax.lax.broadcasted_iota(jnp.int32, (R, LW), 0) % NP
    j_idw = jax.lax.broadcasted_iota(jnp.int32, (R, LW), 1) // HID
    mask_w = (j_idw < N_PART) & (j_idw != i_idw)

    def pair_geom(c):
        diffs = []
        radial = None
        for d in range(DIM):
            cl = jnp.swapaxes(c[d].reshape(BB, NP, 1), 1, 2)
            cj = jnp.broadcast_to(cl, (BB, NP, NP)).reshape(R, NP)
            dd = jnp.broadcast_to(c[d], (R, NP)) - cj
            diffs.append(dd)
            radial = dd * dd if radial is None else radial + dd * dd
        return diffs, radial

    d0, ea_j = pair_geom(crd)

    for l in range(N_LAYERS):
        if l == 0:
            diffs, radial_j = d0, ea_j
        else:
            diffs, radial_j = pair_geom(crd)
        inv = 1.0 / (jnp.sqrt(radial_j + 1e-8) + 1.0)

        hb = h.astype(bf16)
        hwa_t = jnp.dot(hb, wat_ref[l], preferred_element_type=f32)
        hwb = jnp.dot(hb, wb_ref[l], preferred_element_type=f32)
        hwb3 = hwb.reshape(BB, NP, HID)
        hwb_pk = jnp.concatenate([hwb3[:, j, :] for j in range(NP)],
                                 axis=1) + b1t_ref[l]
        hwb_b = jnp.broadcast_to(hwb_pk[:, None, :],
                                 (BB, NP, LW)).reshape(R, LW)
        e_lin = (hwa_t + hwb_b
                 + jnp.dot(radial_j.astype(bf16), wr_ref[l],
                           preferred_element_type=f32)
                 + jnp.dot(ea_j.astype(bf16), we_ref[l],
                           preferred_element_type=f32))
        m1 = _silu(e_lin)
        m = _silu(jnp.dot(m1.astype(bf16), bdw2_ref[l],
                          preferred_element_type=f32) + b2t_ref[l])
        s1 = _silu(jnp.dot(m.astype(bf16), bdc1_ref[l],
                           preferred_element_type=f32) + c1bt_ref[l])
        scal_j = jnp.dot(s1.astype(bf16), c2s_ref[l],
                         preferred_element_type=f32)
        w = inv * jnp.where(mask_j, scal_j, 0.0)
        for d in range(DIM):
            upd = jnp.sum(diffs[d] * w, axis=1, keepdims=True)
            crd[d] = crd[d] + upd

        if l < N_LAYERS - 1:
            m_masked = jnp.where(mask_w, m, 0.0)
            agg = jnp.dot(m_masked.astype(bf16), summ_ref[...],
                          preferred_element_type=f32)
            n1 = _silu(jnp.dot(hb, wn1h_ref[l], preferred_element_type=f32)
                       + jnp.dot(agg.astype(bf16), wn1a_ref[l],
                                 preferred_element_type=f32)
                       + bn1_ref[l])
            h = h + jnp.dot(n1.astype(bf16), wn2_ref[l],
                            preferred_element_type=f32) + bn2_ref[l]

    nmask = jax.lax.broadcasted_iota(jnp.int32, (1, NP, 1), 1) < N_PART
    xp = jnp.concatenate(crd, axis=1).reshape(BB, NP, DIM)
    vec = xp - x
    vec = vec - jnp.sum(jnp.where(nmask, vec, 0.0), axis=1,
                        keepdims=True) * (1.0 / N_PART)
    c_skip = (SIGMA_DATA ** 2) * (c_in * c_in)
    c_out = t * SIGMA_DATA * c_in
    x0 = c_skip[:, :, None] * xt + c_out[:, :, None] * vec
    x0 = x0 - jnp.sum(jnp.where(nmask, x0, 0.0), axis=1,
                      keepdims=True) * (1.0 / N_PART)
    out_ref[...] = x0


@jax.jit
def kernel(xt, t, params):
    B = xt.shape[0]
    xt_p = jnp.pad(xt, ((0, 0), (0, NP - N_PART), (0, 0)))
    t2 = t[:, None]

    L = params["layers"]
    eye = jnp.eye(NP, dtype=jnp.float32)
    stk = lambda f: jnp.stack([f(lp) for lp in L])
    bcast = jnp.bfloat16
    wat = stk(lambda lp: jnp.tile(lp["edge_w1"][:HID],
                                  (1, NP))).astype(bcast)
    wb = stk(lambda lp: lp["edge_w1"][HID:2 * HID]).astype(bcast)
    wr = stk(lambda lp: jnp.kron(eye, lp["edge_w1"][2 * HID:2 * HID + 1])
             ).astype(bcast)
    we = stk(lambda lp: jnp.kron(eye, lp["edge_w1"][2 * HID + 1:2 * HID + 2])
             ).astype(bcast)
    b1t = stk(lambda lp: jnp.tile(lp["edge_b1"][None], (1, NP)))
    bdw2 = stk(lambda lp: jnp.kron(eye, lp["edge_w2"])).astype(jnp.bfloat16)
    b2t = stk(lambda lp: jnp.tile(lp["edge_b2"][None], (1, NP)))
    bdc1 = stk(lambda lp: jnp.kron(eye, lp["coord_w1"])).astype(jnp.bfloat16)
    c1bt = stk(lambda lp: jnp.tile(lp["coord_b1"][None], (1, NP)))
    c2s = stk(lambda lp: jnp.kron(eye, lp["coord_w2"])).astype(jnp.bfloat16)
    summ = jnp.tile(jnp.eye(HID, dtype=jnp.float32),
                    (NP, 1)).astype(jnp.bfloat16)
    wn1h = stk(lambda lp: lp["node_w1"][:HID]).astype(bcast)
    wn1a = stk(lambda lp: lp["node_w1"][HID:]).astype(bcast)
    bn1 = stk(lambda lp: lp["node_b1"][None])
    wn2 = stk(lambda lp: lp["node_w2"]).astype(bcast)
    bn2 = stk(lambda lp: lp["node_b2"][None])
    wsin = params["emb_w"][:HID]
    wcos = params["emb_w"][HID:]
    embb = params["emb_b"][None]

    grid = B // BB
    full = lambda s: pl.BlockSpec(s, lambda b: (0,) * len(s))
    out = pl.pallas_call(
        _fused_kernel,
        grid=(grid,),
        in_specs=[
            pl.BlockSpec((BB, NP, DIM), lambda b: (b, 0, 0)),
            pl.BlockSpec((BB, 1), lambda b: (b, 0)),
            full((HID, HID)), full((HID, HID)), full((1, HID)),
            full((N_LAYERS, HID, LW)), full((N_LAYERS, HID, HID)),
            full((N_LAYERS, NP, LW)), full((N_LAYERS, NP, LW)),
            full((N_LAYERS, 1, LW)),
            full((N_LAYERS, LW, LW)), full((N_LAYERS, 1, LW)),
            full((N_LAYERS, LW, LW)), full((N_LAYERS, 1, LW)),
            full((N_LAYERS, LW, NP)), full((LW, HID)),
            full((N_LAYERS, HID, HID)), full((N_LAYERS, HID, HID)),
            full((N_LAYERS, 1, HID)),
            full((N_LAYERS, HID, HID)), full((N_LAYERS, 1, HID)),
        ],
        out_specs=pl.BlockSpec((BB, NP, DIM), lambda b: (b, 0, 0)),
        out_shape=jax.ShapeDtypeStruct((B, NP, DIM), jnp.float32),
    )(xt_p, t2, wsin, wcos, embb, wat, wb, wr, we, b1t,
      bdw2, b2t, bdc1, c1bt, c2s, summ, wn1h, wn1a, bn1, wn2, bn2)
    return out[:, :N_PART, :]


# speedup vs baseline: 1.0077x; 1.0018x over previous
import jax
import jax.numpy as jnp
import numpy as np
from jax.experimental import pallas as pl

N_PART = 13
NP = 16
DIM = 3
HID = 32
LW = NP * HID
N_LAYERS = 4
SIGMA_DATA = 0.68
BATCH = 4096
BB = 64


def _silu(x):
    return x * (0.5 * jnp.tanh(0.5 * x) + 0.5)


def _fused_kernel(xt_ref, t_ref, wsin_ref, wcos_ref, embb_ref,
                  wat_ref, wb_ref, wr_ref, we_ref, b1t_ref,
                  bdw2_ref, b2t_ref, bdc1_ref, c1bt_ref, c2s_ref, summ_ref,
                  wn1h_ref, wn1a_ref, bn1_ref, wn2_ref, bn2_ref,
                  out_ref):
    f32 = jnp.float32
    bf16 = jnp.bfloat16
    R = BB * NP
    xt = xt_ref[...]
    t = t_ref[...]

    c_in = jax.lax.rsqrt(t * t + SIGMA_DATA ** 2)
    x = xt * c_in[:, :, None]

    k = jax.lax.broadcasted_iota(jnp.int32, (1, HID), 1).astype(f32)
    freqs = jnp.exp((-np.log(10000.0) / 31.0) * k)
    args = (jnp.log(t) * 0.25) * freqs
    h0 = (jnp.dot(jnp.sin(args), wsin_ref[...], preferred_element_type=f32)
          + jnp.dot(jnp.cos(args), wcos_ref[...], preferred_element_type=f32)
          + embb_ref[...])
    h = jnp.broadcast_to(h0[:, None, :], (BB, NP, HID)).reshape(R, HID)

    crd = [x[:, :, d:d + 1].reshape(R, 1) for d in range(DIM)]

    i_id = jax.lax.broadcasted_iota(jnp.int32, (R, NP), 0) % NP
    j_id = jax.lax.broadcasted_iota(jnp.int32, (R, NP), 1)
    mask_j = (j_id < N_PART) & (j_id != i_id)
    i_idw = jax.lax.broadcasted_iota(jnp.int32, (R, LW), 0) % NP
    j_idw = jax.lax.broadcasted_iota(jnp.int32, (R, LW), 1) // HID
    mask_w = (j_idw < N_PART) & (j_idw != i_idw)

    def pair_geom(c):
        diffs = []
        radial = None
        for d in range(DIM):
            cl = jnp.swapaxes(c[d].reshape(BB, NP, 1), 1, 2)
            cj = jnp.broadcast_to(cl, (BB, NP, NP)).reshape(R, NP)
            dd = jnp.broadcast_to(c[d], (R, NP)) - cj
            diffs.append(dd)
            radial = dd * dd if radial is None else radial + dd * dd
        return diffs, radial

    d0, ea_j = pair_geom(crd)

    for l in range(N_LAYERS):
        if l == 0:
            diffs, radial_j = d0, ea_j
        else:
            diffs, radial_j = pair_geom(crd)
        inv = 1.0 / (jnp.sqrt(radial_j + 1e-8) + 1.0)

        hb = h.astype(bf16)
        hwa_t = jnp.dot(hb, wat_ref[l], preferred_element_type=f32)
        hwb = jnp.dot(hb, wb_ref[l], preferred_element_type=f32)
        hwb3 = hwb.reshape(BB, NP, HID)
        hwb_pk = jnp.concatenate([hwb3[:, j, :] for j in range(NP)],
                                 axis=1) + b1t_ref[l]
        hwb_b = jnp.broadcast_to(hwb_pk[:, None, :],
                                 (BB, NP, LW)).reshape(R, LW)
        e_lin = (hwa_t + hwb_b
                 + jnp.dot(radial_j.astype(bf16), wr_ref[l],
                           preferred_element_type=f32)
                 + jnp.dot(ea_j.astype(bf16), we_ref[l],
                           preferred_element_type=f32))
        m1 = _silu(e_lin)
        m = _silu(jnp.dot(m1.astype(bf16), bdw2_ref[l],
                          preferred_element_type=f32) + b2t_ref[l])
        s1 = _silu(jnp.dot(m.astype(bf16), bdc1_ref[l],
                           preferred_element_type=f32) + c1bt_ref[l])
        scal_j = jnp.dot(s1, c2s_ref[l].astype(f32),
                         preferred_element_type=f32)
        w = inv * jnp.where(mask_j, scal_j, 0.0)
        for d in range(DIM):
            upd = jnp.sum(diffs[d] * w, axis=1, keepdims=True)
            crd[d] = crd[d] + upd

        if l < N_LAYERS - 1:
            m_masked = jnp.where(mask_w, m, 0.0)
            agg = jnp.dot(m_masked, summ_ref[...].astype(f32),
                          preferred_element_type=f32)
            n1 = _silu(jnp.dot(hb, wn1h_ref[l], preferred_element_type=f32)
                       + jnp.dot(agg.astype(bf16), wn1a_ref[l],
                                 preferred_element_type=f32)
                       + bn1_ref[l])
            h = h + jnp.dot(n1.astype(bf16), wn2_ref[l],
                            preferred_element_type=f32) + bn2_ref[l]

    nmask = jax.lax.broadcasted_iota(jnp.int32, (1, NP, 1), 1) < N_PART
    xp = jnp.concatenate(crd, axis=1).reshape(BB, NP, DIM)
    vec = xp - x
    vec = vec - jnp.sum(jnp.where(nmask, vec, 0.0), axis=1,
                        keepdims=True) * (1.0 / N_PART)
    c_skip = (SIGMA_DATA ** 2) * (c_in * c_in)
    c_out = t * SIGMA_DATA * c_in
    x0 = c_skip[:, :, None] * xt + c_out[:, :, None] * vec
    x0 = x0 - jnp.sum(jnp.where(nmask, x0, 0.0), axis=1,
                      keepdims=True) * (1.0 / N_PART)
    out_ref[...] = x0


@jax.jit
def kernel(xt, t, params):
    B = xt.shape[0]
    xt_p = jnp.pad(xt, ((0, 0), (0, NP - N_PART), (0, 0)))
    t2 = t[:, None]

    L = params["layers"]
    eye = jnp.eye(NP, dtype=jnp.float32)
    stk = lambda f: jnp.stack([f(lp) for lp in L])
    bcast = jnp.bfloat16
    wat = stk(lambda lp: jnp.tile(lp["edge_w1"][:HID],
                                  (1, NP))).astype(bcast)
    wb = stk(lambda lp: lp["edge_w1"][HID:2 * HID]).astype(bcast)
    wr = stk(lambda lp: jnp.kron(eye, lp["edge_w1"][2 * HID:2 * HID + 1])
             ).astype(bcast)
    we = stk(lambda lp: jnp.kron(eye, lp["edge_w1"][2 * HID + 1:2 * HID + 2])
             ).astype(bcast)
    b1t = stk(lambda lp: jnp.tile(lp["edge_b1"][None], (1, NP)))
    bdw2 = stk(lambda lp: jnp.kron(eye, lp["edge_w2"])).astype(jnp.bfloat16)
    b2t = stk(lambda lp: jnp.tile(lp["edge_b2"][None], (1, NP)))
    bdc1 = stk(lambda lp: jnp.kron(eye, lp["coord_w1"])).astype(jnp.bfloat16)
    c1bt = stk(lambda lp: jnp.tile(lp["coord_b1"][None], (1, NP)))
    c2s = stk(lambda lp: jnp.kron(eye, lp["coord_w2"])).astype(jnp.bfloat16)
    summ = jnp.tile(jnp.eye(HID, dtype=jnp.float32),
                    (NP, 1)).astype(jnp.bfloat16)
    wn1h = stk(lambda lp: lp["node_w1"][:HID]).astype(bcast)
    wn1a = stk(lambda lp: lp["node_w1"][HID:]).astype(bcast)
    bn1 = stk(lambda lp: lp["node_b1"][None])
    wn2 = stk(lambda lp: lp["node_w2"]).astype(bcast)
    bn2 = stk(lambda lp: lp["node_b2"][None])
    wsin = params["emb_w"][:HID]
    wcos = params["emb_w"][HID:]
    embb = params["emb_b"][None]

    grid = B // BB
    full = lambda s: pl.BlockSpec(s, lambda b: (0,) * len(s))
    out = pl.pallas_call(
        _fused_kernel,
        grid=(grid,),
        in_specs=[
            pl.BlockSpec((BB, NP, DIM), lambda b: (b, 0, 0)),
            pl.BlockSpec((BB, 1), lambda b: (b, 0)),
            full((HID, HID)), full((HID, HID)), full((1, HID)),
            full((N_LAYERS, HID, LW)), full((N_LAYERS, HID, HID)),
            full((N_LAYERS, NP, LW)), full((N_LAYERS, NP, LW)),
            full((N_LAYERS, 1, LW)),
            full((N_LAYERS, LW, LW)), full((N_LAYERS, 1, LW)),
            full((N_LAYERS, LW, LW)), full((N_LAYERS, 1, LW)),
            full((N_LAYERS, LW, NP)), full((LW, HID)),
            full((N_LAYERS, HID, HID)), full((N_LAYERS, HID, HID)),
            full((N_LAYERS, 1, HID)),
            full((N_LAYERS, HID, HID)), full((N_LAYERS, 1, HID)),
        ],
        out_specs=pl.BlockSpec((BB, NP, DIM), lambda b: (b, 0, 0)),
        out_shape=jax.ShapeDtypeStruct((B, NP, DIM), jnp.float32),
    )(xt_p, t2, wsin, wcos, embb, wat, wb, wr, we, b1t,
      bdw2, b2t, bdc1, c1bt, c2s, summ, wn1h, wn1a, bn1, wn2, bn2)
    return out[:, :N_PART, :]


# speedup vs baseline: 1.0297x; 1.0218x over previous
import jax
import jax.numpy as jnp
import numpy as np
from jax.experimental import pallas as pl

N_PART = 13
NP = 16
DIM = 3
HID = 32
LW = NP * HID
N_LAYERS = 4
SIGMA_DATA = 0.68
BATCH = 4096
BB = 64


def _silu(x):
    s = 0.5 * x
    return s + s * jnp.tanh(s)


def _fused_kernel(xt_ref, t_ref, wsin_ref, wcos_ref, embb_ref,
                  wat_ref, wb_ref, wr_ref, we_ref, b1t_ref,
                  bdw2_ref, b2t_ref, bdc1_ref, c1bt_ref, c2s_ref, summ_ref,
                  wn1h_ref, wn1a_ref, bn1_ref, wn2_ref, bn2_ref,
                  out_ref):
    f32 = jnp.float32
    bf16 = jnp.bfloat16
    R = BB * NP
    xt = xt_ref[...]
    t = t_ref[...]

    c_in = jax.lax.rsqrt(t * t + SIGMA_DATA ** 2)
    x = xt * c_in[:, :, None]

    k = jax.lax.broadcasted_iota(jnp.int32, (1, HID), 1).astype(f32)
    freqs = jnp.exp((-np.log(10000.0) / 31.0) * k)
    args = (jnp.log(t) * 0.25) * freqs
    h0 = (jnp.dot(jnp.sin(args), wsin_ref[...], preferred_element_type=f32)
          + jnp.dot(jnp.cos(args), wcos_ref[...], preferred_element_type=f32)
          + embb_ref[...])
    h = jnp.broadcast_to(h0[:, None, :], (BB, NP, HID)).reshape(R, HID)

    crd = [x[:, :, d:d + 1].reshape(R, 1) for d in range(DIM)]

    i_id = jax.lax.broadcasted_iota(jnp.int32, (R, NP), 0) % NP
    j_id = jax.lax.broadcasted_iota(jnp.int32, (R, NP), 1)
    mask_j = (j_id < N_PART) & (j_id != i_id)
    i_idw = jax.lax.broadcasted_iota(jnp.int32, (R, LW), 0) % NP
    j_idw = jax.lax.broadcasted_iota(jnp.int32, (R, LW), 1) // HID
    mask_w = (j_idw < N_PART) & (j_idw != i_idw)

    def pair_geom(c):
        diffs = []
        radial = None
        for d in range(DIM):
            cl = jnp.swapaxes(c[d].reshape(BB, NP, 1), 1, 2)
            cj = jnp.broadcast_to(cl, (BB, NP, NP)).reshape(R, NP)
            dd = jnp.broadcast_to(c[d], (R, NP)) - cj
            diffs.append(dd)
            radial = dd * dd if radial is None else radial + dd * dd
        return diffs, radial

    d0, ea_j = pair_geom(crd)

    for l in range(N_LAYERS):
        if l == 0:
            diffs, radial_j = d0, ea_j
        else:
            diffs, radial_j = pair_geom(crd)
        inv = 1.0 / (jnp.sqrt(radial_j + 1e-8) + 1.0)

        hb = h.astype(bf16)
        hwa_t = jnp.dot(hb, wat_ref[l], preferred_element_type=f32)
        hwb = jnp.dot(hb, wb_ref[l], preferred_element_type=f32)
        hwb3 = hwb.reshape(BB, NP, HID)
        hwb_pk = jnp.concatenate([hwb3[:, j, :] for j in range(NP)],
                                 axis=1) + b1t_ref[l]
        hwb_b = jnp.broadcast_to(hwb_pk[:, None, :],
                                 (BB, NP, LW)).reshape(R, LW)
        e_lin = (hwa_t + hwb_b
                 + jnp.dot(radial_j.astype(bf16), wr_ref[l],
                           preferred_element_type=f32)
                 + jnp.dot(ea_j.astype(bf16), we_ref[l],
                           preferred_element_type=f32))
        m1 = _silu(e_lin)
        m = _silu(jnp.dot(m1.astype(bf16), bdw2_ref[l],
                          preferred_element_type=f32) + b2t_ref[l])
        s1 = _silu(jnp.dot(m.astype(bf16), bdc1_ref[l],
                           preferred_element_type=f32) + c1bt_ref[l])
        scal_j = jnp.dot(s1, c2s_ref[l].astype(f32),
                         preferred_element_type=f32)
        w = inv * jnp.where(mask_j, scal_j, 0.0)
        for d in range(DIM):
            upd = jnp.sum(diffs[d] * w, axis=1, keepdims=True)
            crd[d] = crd[d] + upd

        if l < N_LAYERS - 1:
            m_masked = jnp.where(mask_w, m, 0.0)
            agg = jnp.dot(m_masked, summ_ref[...].astype(f32),
                          preferred_element_type=f32)
            n1 = _silu(jnp.dot(hb, wn1h_ref[l], preferred_element_type=f32)
                       + jnp.dot(agg.astype(bf16), wn1a_ref[l],
                                 preferred_element_type=f32)
                       + bn1_ref[l])
            h = h + jnp.dot(n1.astype(bf16), wn2_ref[l],
                            preferred_element_type=f32) + bn2_ref[l]

    nmask = jax.lax.broadcasted_iota(jnp.int32, (1, NP, 1), 1) < N_PART
    xp = jnp.concatenate(crd, axis=1).reshape(BB, NP, DIM)
    vec = xp - x
    vec = vec - jnp.sum(jnp.where(nmask, vec, 0.0), axis=1,
                        keepdims=True) * (1.0 / N_PART)
    c_skip = (SIGMA_DATA ** 2) * (c_in * c_in)
    c_out = t * SIGMA_DATA * c_in
    x0 = c_skip[:, :, None] * xt + c_out[:, :, None] * vec
    x0 = x0 - jnp.sum(jnp.where(nmask, x0, 0.0), axis=1,
                      keepdims=True) * (1.0 / N_PART)
    out_ref[...] = x0


@jax.jit
def kernel(xt, t, params):
    B = xt.shape[0]
    xt_p = jnp.pad(xt, ((0, 0), (0, NP - N_PART), (0, 0)))
    t2 = t[:, None]

    L = params["layers"]
    eye = jnp.eye(NP, dtype=jnp.float32)
    stk = lambda f: jnp.stack([f(lp) for lp in L])
    bcast = jnp.bfloat16
    wat = stk(lambda lp: jnp.tile(lp["edge_w1"][:HID],
                                  (1, NP))).astype(bcast)
    wb = stk(lambda lp: lp["edge_w1"][HID:2 * HID]).astype(bcast)
    wr = stk(lambda lp: jnp.kron(eye, lp["edge_w1"][2 * HID:2 * HID + 1])
             ).astype(bcast)
    we = stk(lambda lp: jnp.kron(eye, lp["edge_w1"][2 * HID + 1:2 * HID + 2])
             ).astype(bcast)
    b1t = stk(lambda lp: jnp.tile(lp["edge_b1"][None], (1, NP)))
    bdw2 = stk(lambda lp: jnp.kron(eye, lp["edge_w2"])).astype(jnp.bfloat16)
    b2t = stk(lambda lp: jnp.tile(lp["edge_b2"][None], (1, NP)))
    bdc1 = stk(lambda lp: jnp.kron(eye, lp["coord_w1"])).astype(jnp.bfloat16)
    c1bt = stk(lambda lp: jnp.tile(lp["coord_b1"][None], (1, NP)))
    c2s = stk(lambda lp: jnp.kron(eye, lp["coord_w2"])).astype(jnp.bfloat16)
    summ = jnp.tile(jnp.eye(HID, dtype=jnp.float32),
                    (NP, 1)).astype(jnp.bfloat16)
    wn1h = stk(lambda lp: lp["node_w1"][:HID]).astype(bcast)
    wn1a = stk(lambda lp: lp["node_w1"][HID:]).astype(bcast)
    bn1 = stk(lambda lp: lp["node_b1"][None])
    wn2 = stk(lambda lp: lp["node_w2"]).astype(bcast)
    bn2 = stk(lambda lp: lp["node_b2"][None])
    wsin = params["emb_w"][:HID]
    wcos = params["emb_w"][HID:]
    embb = params["emb_b"][None]

    grid = B // BB
    full = lambda s: pl.BlockSpec(s, lambda b: (0,) * len(s))
    out = pl.pallas_call(
        _fused_kernel,
        grid=(grid,),
        in_specs=[
            pl.BlockSpec((BB, NP, DIM), lambda b: (b, 0, 0)),
            pl.BlockSpec((BB, 1), lambda b: (b, 0)),
            full((HID, HID)), full((HID, HID)), full((1, HID)),
            full((N_LAYERS, HID, LW)), full((N_LAYERS, HID, HID)),
            full((N_LAYERS, NP, LW)), full((N_LAYERS, NP, LW)),
            full((N_LAYERS, 1, LW)),
            full((N_LAYERS, LW, LW)), full((N_LAYERS, 1, LW)),
            full((N_LAYERS, LW, LW)), full((N_LAYERS, 1, LW)),
            full((N_LAYERS, LW, NP)), full((LW, HID)),
            full((N_LAYERS, HID, HID)), full((N_LAYERS, HID, HID)),
            full((N_LAYERS, 1, HID)),
            full((N_LAYERS, HID, HID)), full((N_LAYERS, 1, HID)),
        ],
        out_specs=pl.BlockSpec((BB, NP, DIM), lambda b: (b, 0, 0)),
        out_shape=jax.ShapeDtypeStruct((B, NP, DIM), jnp.float32),
    )(xt_p, t2, wsin, wcos, embb, wat, wb, wr, we, b1t,
      bdw2, b2t, bdc1, c1bt, c2s, summ, wn1h, wn1a, bn1, wn2, bn2)
    return out[:, :N_PART, :]


# speedup vs baseline: 1.0380x; 1.0081x over previous
import jax
import jax.numpy as jnp
import numpy as np
from jax.experimental import pallas as pl

N_PART = 13
NP = 16
DIM = 3
HID = 32
LW = NP * HID
N_LAYERS = 4
SIGMA_DATA = 0.68
BATCH = 4096
BB = 64


def _silu(x):
    s = 0.5 * x
    return s + s * jnp.tanh(s)


def _fused_kernel(xt_ref, t_ref, wsin_ref, wcos_ref, embb_ref,
                  wat_ref, wb_ref, wbt_ref, wre0_ref, wr_ref, we_ref, b1t_ref,
                  bdw2_ref, b2t_ref, bdc1_ref, c1bt_ref, c2s_ref, summ_ref,
                  wn1h_ref, wn1a_ref, bn1_ref, wn2_ref, bn2_ref,
                  out_ref):
    f32 = jnp.float32
    bf16 = jnp.bfloat16
    R = BB * NP
    xt = xt_ref[...]
    t = t_ref[...]

    c_in = jax.lax.rsqrt(t * t + SIGMA_DATA ** 2)
    x = xt * c_in[:, :, None]

    k = jax.lax.broadcasted_iota(jnp.int32, (1, HID), 1).astype(f32)
    freqs = jnp.exp((-np.log(10000.0) / 31.0) * k)
    args = (jnp.log(t) * 0.25) * freqs
    h0 = (jnp.dot(jnp.sin(args), wsin_ref[...], preferred_element_type=f32)
          + jnp.dot(jnp.cos(args), wcos_ref[...], preferred_element_type=f32)
          + embb_ref[...])
    h = jnp.broadcast_to(h0[:, None, :], (BB, NP, HID)).reshape(R, HID)

    crd = [x[:, :, d:d + 1].reshape(R, 1) for d in range(DIM)]

    i_id = jax.lax.broadcasted_iota(jnp.int32, (R, NP), 0) % NP
    j_id = jax.lax.broadcasted_iota(jnp.int32, (R, NP), 1)
    mask_j = (j_id < N_PART) & (j_id != i_id)
    i_idw = jax.lax.broadcasted_iota(jnp.int32, (R, LW), 0) % NP
    j_idw = jax.lax.broadcasted_iota(jnp.int32, (R, LW), 1) // HID
    mask_w = (j_idw < N_PART) & (j_idw != i_idw)

    def pair_geom(c):
        diffs = []
        radial = None
        for d in range(DIM):
            cl = jnp.swapaxes(c[d].reshape(BB, NP, 1), 1, 2)
            cj = jnp.broadcast_to(cl, (BB, NP, NP)).reshape(R, NP)
            dd = jnp.broadcast_to(c[d], (R, NP)) - cj
            diffs.append(dd)
            radial = dd * dd if radial is None else radial + dd * dd
        return diffs, radial

    d0, ea_j = pair_geom(crd)

    for l in range(N_LAYERS):
        if l == 0:
            diffs, radial_j = d0, ea_j
        else:
            diffs, radial_j = pair_geom(crd)
        inv = 1.0 / (jnp.sqrt(radial_j + 1e-8) + 1.0)

        hb = h.astype(bf16)
        hwa_t = jnp.dot(hb, wat_ref[l], preferred_element_type=f32)
        if l == 0:
            hwb_pk = jnp.dot(h0.astype(bf16), wbt_ref[...],
                             preferred_element_type=f32) + b1t_ref[l]
        else:
            hwb = jnp.dot(hb, wb_ref[l], preferred_element_type=f32)
            hwb3 = hwb.reshape(BB, NP, HID)
            hwb_pk = jnp.concatenate([hwb3[:, j, :] for j in range(NP)],
                                     axis=1) + b1t_ref[l]
        hwb_b = jnp.broadcast_to(hwb_pk[:, None, :],
                                 (BB, NP, LW)).reshape(R, LW)
        if l == 0:
            spread = jnp.dot(radial_j.astype(bf16), wre0_ref[...],
                             preferred_element_type=f32)
        else:
            spread = (jnp.dot(radial_j.astype(bf16), wr_ref[l],
                              preferred_element_type=f32)
                      + jnp.dot(ea_j.astype(bf16), we_ref[l],
                                preferred_element_type=f32))
        e_lin = hwa_t + hwb_b + spread
        m1 = _silu(e_lin)
        m = _silu(jnp.dot(m1.astype(bf16), bdw2_ref[l],
                          preferred_element_type=f32) + b2t_ref[l])
        s1 = _silu(jnp.dot(m.astype(bf16), bdc1_ref[l],
                           preferred_element_type=f32) + c1bt_ref[l])
        scal_j = jnp.dot(s1, c2s_ref[l].astype(f32),
                         preferred_element_type=f32)
        w = inv * jnp.where(mask_j, scal_j, 0.0)
        for d in range(DIM):
            upd = jnp.sum(diffs[d] * w, axis=1, keepdims=True)
            crd[d] = crd[d] + upd

        if l < N_LAYERS - 1:
            m_masked = jnp.where(mask_w, m, 0.0)
            agg = jnp.dot(m_masked, summ_ref[...].astype(f32),
                          preferred_element_type=f32)
            n1 = _silu(jnp.dot(hb, wn1h_ref[l], preferred_element_type=f32)
                       + jnp.dot(agg.astype(bf16), wn1a_ref[l],
                                 preferred_element_type=f32)
                       + bn1_ref[l])
            h = h + jnp.dot(n1.astype(bf16), wn2_ref[l],
                            preferred_element_type=f32) + bn2_ref[l]

    nmask = jax.lax.broadcasted_iota(jnp.int32, (1, NP, 1), 1) < N_PART
    xp = jnp.concatenate(crd, axis=1).reshape(BB, NP, DIM)
    vec = xp - x
    vec = vec - jnp.sum(jnp.where(nmask, vec, 0.0), axis=1,
                        keepdims=True) * (1.0 / N_PART)
    c_skip = (SIGMA_DATA ** 2) * (c_in * c_in)
    c_out = t * SIGMA_DATA * c_in
    x0 = c_skip[:, :, None] * xt + c_out[:, :, None] * vec
    x0 = x0 - jnp.sum(jnp.where(nmask, x0, 0.0), axis=1,
                      keepdims=True) * (1.0 / N_PART)
    out_ref[...] = x0


@jax.jit
def kernel(xt, t, params):
    B = xt.shape[0]
    xt_p = jnp.pad(xt, ((0, 0), (0, NP - N_PART), (0, 0)))
    t2 = t[:, None]

    L = params["layers"]
    eye = jnp.eye(NP, dtype=jnp.float32)
    stk = lambda f: jnp.stack([f(lp) for lp in L])
    bcast = jnp.bfloat16
    wat = stk(lambda lp: jnp.tile(lp["edge_w1"][:HID],
                                  (1, NP))).astype(bcast)
    wb = stk(lambda lp: lp["edge_w1"][HID:2 * HID]).astype(bcast)
    wr = stk(lambda lp: jnp.kron(eye, lp["edge_w1"][2 * HID:2 * HID + 1])
             ).astype(bcast)
    we = stk(lambda lp: jnp.kron(eye, lp["edge_w1"][2 * HID + 1:2 * HID + 2])
             ).astype(bcast)
    wbt = jnp.tile(L[0]["edge_w1"][HID:2 * HID], (1, NP)).astype(bcast)
    wre0 = jnp.kron(eye, L[0]["edge_w1"][2 * HID:2 * HID + 1]
                    + L[0]["edge_w1"][2 * HID + 1:2 * HID + 2]).astype(bcast)
    b1t = stk(lambda lp: jnp.tile(lp["edge_b1"][None], (1, NP)))
    bdw2 = stk(lambda lp: jnp.kron(eye, lp["edge_w2"])).astype(jnp.bfloat16)
    b2t = stk(lambda lp: jnp.tile(lp["edge_b2"][None], (1, NP)))
    bdc1 = stk(lambda lp: jnp.kron(eye, lp["coord_w1"])).astype(jnp.bfloat16)
    c1bt = stk(lambda lp: jnp.tile(lp["coord_b1"][None], (1, NP)))
    c2s = stk(lambda lp: jnp.kron(eye, lp["coord_w2"])).astype(jnp.bfloat16)
    summ = jnp.tile(jnp.eye(HID, dtype=jnp.float32),
                    (NP, 1)).astype(jnp.bfloat16)
    wn1h = stk(lambda lp: lp["node_w1"][:HID]).astype(bcast)
    wn1a = stk(lambda lp: lp["node_w1"][HID:]).astype(bcast)
    bn1 = stk(lambda lp: lp["node_b1"][None])
    wn2 = stk(lambda lp: lp["node_w2"]).astype(bcast)
    bn2 = stk(lambda lp: lp["node_b2"][None])
    wsin = params["emb_w"][:HID]
    wcos = params["emb_w"][HID:]
    embb = params["emb_b"][None]

    grid = B // BB
    full = lambda s: pl.BlockSpec(s, lambda b: (0,) * len(s))
    out = pl.pallas_call(
        _fused_kernel,
        grid=(grid,),
        in_specs=[
            pl.BlockSpec((BB, NP, DIM), lambda b: (b, 0, 0)),
            pl.BlockSpec((BB, 1), lambda b: (b, 0)),
            full((HID, HID)), full((HID, HID)), full((1, HID)),
            full((N_LAYERS, HID, LW)), full((N_LAYERS, HID, HID)),
            full((HID, LW)), full((NP, LW)),
            full((N_LAYERS, NP, LW)), full((N_LAYERS, NP, LW)),
            full((N_LAYERS, 1, LW)),
            full((N_LAYERS, LW, LW)), full((N_LAYERS, 1, LW)),
            full((N_LAYERS, LW, LW)), full((N_LAYERS, 1, LW)),
            full((N_LAYERS, LW, NP)), full((LW, HID)),
            full((N_LAYERS, HID, HID)), full((N_LAYERS, HID, HID)),
            full((N_LAYERS, 1, HID)),
            full((N_LAYERS, HID, HID)), full((N_LAYERS, 1, HID)),
        ],
        out_specs=pl.BlockSpec((BB, NP, DIM), lambda b: (b, 0, 0)),
        out_shape=jax.ShapeDtypeStruct((B, NP, DIM), jnp.float32),
    )(xt_p, t2, wsin, wcos, embb, wat, wb, wbt, wre0, wr, we, b1t,
      bdw2, b2t, bdc1, c1bt, c2s, summ, wn1h, wn1a, bn1, wn2, bn2)
    return out[:, :N_PART, :]


# speedup vs baseline: 1.0549x; 1.0163x over previous
import jax
import jax.numpy as jnp
import numpy as np
from jax.experimental import pallas as pl

N_PART = 13
NP = 16
DIM = 3
HID = 32
LW = NP * HID
N_LAYERS = 4
SIGMA_DATA = 0.68
BATCH = 4096
BB = 64


def _silu(x):
    s = 0.5 * x
    return s + s * jnp.tanh(s)


def _fused_kernel(xt_ref, t_ref, wsin_ref, wcos_ref, embb_ref,
                  wat_ref, wb_ref, wbt_ref, wre0_ref, wr_ref, we_ref, b1t_ref,
                  bdw2_ref, b2t_ref, bdc1_ref, c1bt_ref, c2s_ref, summ_ref,
                  wn1h_ref, wn1a_ref, bn1_ref, wn2_ref, bn2_ref,
                  out_ref):
    f32 = jnp.float32
    bf16 = jnp.bfloat16
    R = BB * NP
    xt = xt_ref[...]
    t = t_ref[...]

    c_in = jax.lax.rsqrt(t * t + SIGMA_DATA ** 2)
    x = xt * c_in[:, :, None]

    k = jax.lax.broadcasted_iota(jnp.int32, (1, HID), 1).astype(f32)
    freqs = jnp.exp((-np.log(10000.0) / 31.0) * k)
    args = (jnp.log(t) * 0.25) * freqs
    h0 = (jnp.dot(jnp.sin(args), wsin_ref[...], preferred_element_type=f32)
          + jnp.dot(jnp.cos(args), wcos_ref[...], preferred_element_type=f32)
          + embb_ref[...])
    h = jnp.broadcast_to(h0[:, None, :], (BB, NP, HID)).reshape(R, HID)

    crd = [x[:, :, d:d + 1].reshape(R, 1) for d in range(DIM)]

    i_id = jax.lax.broadcasted_iota(jnp.int32, (R, NP), 0) % NP
    j_id = jax.lax.broadcasted_iota(jnp.int32, (R, NP), 1)
    mask_j = (j_id < N_PART) & (j_id != i_id)
    i_idw = jax.lax.broadcasted_iota(jnp.int32, (R, LW), 0) % NP
    j_idw = jax.lax.broadcasted_iota(jnp.int32, (R, LW), 1) // HID
    mask_w = (j_idw < N_PART) & (j_idw != i_idw)

    def pair_geom(c):
        diffs = []
        radial = None
        for d in range(DIM):
            cl = jnp.swapaxes(c[d].reshape(BB, NP, 1), 1, 2)
            cj = jnp.broadcast_to(cl, (BB, NP, NP)).reshape(R, NP)
            dd = jnp.broadcast_to(c[d], (R, NP)) - cj
            diffs.append(dd)
            radial = dd * dd if radial is None else radial + dd * dd
        return diffs, radial

    d0, ea_j = pair_geom(crd)

    for l in range(N_LAYERS):
        if l == 0:
            diffs, radial_j = d0, ea_j
        else:
            diffs, radial_j = pair_geom(crd)
        inv = 1.0 / (jnp.sqrt(radial_j + 1e-8) + 1.0)

        hb = h.astype(bf16)
        hwa_t = jnp.dot(hb, wat_ref[l], preferred_element_type=f32)
        if l == 0:
            hwb_pk = jnp.dot(h0.astype(bf16), wbt_ref[...],
                             preferred_element_type=f32) + b1t_ref[l]
        else:
            hwb = jnp.dot(hb, wb_ref[l], preferred_element_type=f32)
            hwb3 = hwb.reshape(BB, NP, HID)
            hwb_pk = jnp.concatenate([hwb3[:, j, :] for j in range(NP)],
                                     axis=1) + b1t_ref[l]
        hwb_b = jnp.broadcast_to(hwb_pk[:, None, :],
                                 (BB, NP, LW)).reshape(R, LW)
        if l == 0:
            spread = jnp.dot(radial_j.astype(bf16), wre0_ref[...],
                             preferred_element_type=f32)
        else:
            spread = (jnp.dot(radial_j.astype(bf16), wr_ref[l],
                              preferred_element_type=f32)
                      + jnp.dot(ea_j.astype(bf16), we_ref[l],
                                preferred_element_type=f32))
        e_lin = hwa_t + hwb_b + spread
        m1 = _silu(e_lin.astype(bf16))
        m = _silu(jnp.dot(m1, bdw2_ref[l],
                          preferred_element_type=f32) + b2t_ref[l])
        s1 = _silu(jnp.dot(m.astype(bf16), bdc1_ref[l],
                           preferred_element_type=f32) + c1bt_ref[l])
        scal_j = jnp.dot(s1, c2s_ref[l].astype(f32),
                         preferred_element_type=f32)
        w = inv * jnp.where(mask_j, scal_j, 0.0)
        for d in range(DIM):
            upd = jnp.sum(diffs[d] * w, axis=1, keepdims=True)
            crd[d] = crd[d] + upd

        if l < N_LAYERS - 1:
            m_masked = jnp.where(mask_w, m, 0.0)
            agg = jnp.dot(m_masked, summ_ref[...].astype(f32),
                          preferred_element_type=f32)
            n1 = _silu(jnp.dot(hb, wn1h_ref[l], preferred_element_type=f32)
                       + jnp.dot(agg.astype(bf16), wn1a_ref[l],
                                 preferred_element_type=f32)
                       + bn1_ref[l])
            h = h + jnp.dot(n1.astype(bf16), wn2_ref[l],
                            preferred_element_type=f32) + bn2_ref[l]

    nmask = jax.lax.broadcasted_iota(jnp.int32, (1, NP, 1), 1) < N_PART
    xp = jnp.concatenate(crd, axis=1).reshape(BB, NP, DIM)
    vec = xp - x
    vec = vec - jnp.sum(jnp.where(nmask, vec, 0.0), axis=1,
                        keepdims=True) * (1.0 / N_PART)
    c_skip = (SIGMA_DATA ** 2) * (c_in * c_in)
    c_out = t * SIGMA_DATA * c_in
    x0 = c_skip[:, :, None] * xt + c_out[:, :, None] * vec
    x0 = x0 - jnp.sum(jnp.where(nmask, x0, 0.0), axis=1,
                      keepdims=True) * (1.0 / N_PART)
    out_ref[...] = x0


@jax.jit
def kernel(xt, t, params):
    B = xt.shape[0]
    xt_p = jnp.pad(xt, ((0, 0), (0, NP - N_PART), (0, 0)))
    t2 = t[:, None]

    L = params["layers"]
    eye = jnp.eye(NP, dtype=jnp.float32)
    stk = lambda f: jnp.stack([f(lp) for lp in L])
    bcast = jnp.bfloat16
    wat = stk(lambda lp: jnp.tile(lp["edge_w1"][:HID],
                                  (1, NP))).astype(bcast)
    wb = stk(lambda lp: lp["edge_w1"][HID:2 * HID]).astype(bcast)
    wr = stk(lambda lp: jnp.kron(eye, lp["edge_w1"][2 * HID:2 * HID + 1])
             ).astype(bcast)
    we = stk(lambda lp: jnp.kron(eye, lp["edge_w1"][2 * HID + 1:2 * HID + 2])
             ).astype(bcast)
    wbt = jnp.tile(L[0]["edge_w1"][HID:2 * HID], (1, NP)).astype(bcast)
    wre0 = jnp.kron(eye, L[0]["edge_w1"][2 * HID:2 * HID + 1]
                    + L[0]["edge_w1"][2 * HID + 1:2 * HID + 2]).astype(bcast)
    b1t = stk(lambda lp: jnp.tile(lp["edge_b1"][None], (1, NP)))
    bdw2 = stk(lambda lp: jnp.kron(eye, lp["edge_w2"])).astype(jnp.bfloat16)
    b2t = stk(lambda lp: jnp.tile(lp["edge_b2"][None], (1, NP)))
    bdc1 = stk(lambda lp: jnp.kron(eye, lp["coord_w1"])).astype(jnp.bfloat16)
    c1bt = stk(lambda lp: jnp.tile(lp["coord_b1"][None], (1, NP)))
    c2s = stk(lambda lp: jnp.kron(eye, lp["coord_w2"])).astype(jnp.bfloat16)
    summ = jnp.tile(jnp.eye(HID, dtype=jnp.float32),
                    (NP, 1)).astype(jnp.bfloat16)
    wn1h = stk(lambda lp: lp["node_w1"][:HID]).astype(bcast)
    wn1a = stk(lambda lp: lp["node_w1"][HID:]).astype(bcast)
    bn1 = stk(lambda lp: lp["node_b1"][None])
    wn2 = stk(lambda lp: lp["node_w2"]).astype(bcast)
    bn2 = stk(lambda lp: lp["node_b2"][None])
    wsin = params["emb_w"][:HID]
    wcos = params["emb_w"][HID:]
    embb = params["emb_b"][None]

    grid = B // BB
    full = lambda s: pl.BlockSpec(s, lambda b: (0,) * len(s))
    out = pl.pallas_call(
        _fused_kernel,
        grid=(grid,),
        in_specs=[
            pl.BlockSpec((BB, NP, DIM), lambda b: (b, 0, 0)),
            pl.BlockSpec((BB, 1), lambda b: (b, 0)),
            full((HID, HID)), full((HID, HID)), full((1, HID)),
            full((N_LAYERS, HID, LW)), full((N_LAYERS, HID, HID)),
            full((HID, LW)), full((NP, LW)),
            full((N_LAYERS, NP, LW)), full((N_LAYERS, NP, LW)),
            full((N_LAYERS, 1, LW)),
            full((N_LAYERS, LW, LW)), full((N_LAYERS, 1, LW)),
            full((N_LAYERS, LW, LW)), full((N_LAYERS, 1, LW)),
            full((N_LAYERS, LW, NP)), full((LW, HID)),
            full((N_LAYERS, HID, HID)), full((N_LAYERS, HID, HID)),
            full((N_LAYERS, 1, HID)),
            full((N_LAYERS, HID, HID)), full((N_LAYERS, 1, HID)),
        ],
        out_specs=pl.BlockSpec((BB, NP, DIM), lambda b: (b, 0, 0)),
        out_shape=jax.ShapeDtypeStruct((B, NP, DIM), jnp.float32),
    )(xt_p, t2, wsin, wcos, embb, wat, wb, wbt, wre0, wr, we, b1t,
      bdw2, b2t, bdc1, c1bt, c2s, summ, wn1h, wn1a, bn1, wn2, bn2)
    return out[:, :N_PART, :]


# speedup vs baseline: 1.0637x; 1.0083x over previous
import jax
import jax.numpy as jnp
import numpy as np
from jax.experimental import pallas as pl

N_PART = 13
NP = 16
DIM = 3
HID = 32
LW = NP * HID
N_LAYERS = 4
SIGMA_DATA = 0.68
BATCH = 4096
BB = 64


def _silu(x):
    s = 0.5 * x
    return s + s * jnp.tanh(s)


def _fused_kernel(xt_ref, t_ref, wsin_ref, wcos_ref, embb_ref,
                  wat_ref, wb_ref, wbt_ref, wre0_ref, wr_ref, we_ref, b1t_ref,
                  bdw2_ref, b2t_ref, bdc1_ref, c1bt_ref, c2s_ref, summ_ref,
                  wn1h_ref, wn1a_ref, bn1_ref, wn2_ref, bn2_ref,
                  out_ref):
    f32 = jnp.float32
    bf16 = jnp.bfloat16
    R = BB * NP
    xt = xt_ref[...]
    t = t_ref[...]

    c_in = jax.lax.rsqrt(t * t + SIGMA_DATA ** 2)
    x = xt * c_in[:, :, None]

    k = jax.lax.broadcasted_iota(jnp.int32, (1, HID), 1).astype(f32)
    freqs = jnp.exp((-np.log(10000.0) / 31.0) * k)
    args = (jnp.log(t) * 0.25) * freqs
    h0 = (jnp.dot(jnp.sin(args), wsin_ref[...], preferred_element_type=f32)
          + jnp.dot(jnp.cos(args), wcos_ref[...], preferred_element_type=f32)
          + embb_ref[...])
    h = jnp.broadcast_to(h0[:, None, :], (BB, NP, HID)).reshape(R, HID)

    crd = [x[:, :, d:d + 1].reshape(R, 1) for d in range(DIM)]

    i_id = jax.lax.broadcasted_iota(jnp.int32, (R, NP), 0) % NP
    j_id = jax.lax.broadcasted_iota(jnp.int32, (R, NP), 1)
    mask_j = (j_id < N_PART) & (j_id != i_id)
    i_idw = jax.lax.broadcasted_iota(jnp.int32, (R, LW), 0) % NP
    j_idw = jax.lax.broadcasted_iota(jnp.int32, (R, LW), 1) // HID
    mask_w = (j_idw < N_PART) & (j_idw != i_idw)

    def pair_geom(c):
        diffs = []
        radial = None
        for d in range(DIM):
            cl = jnp.swapaxes(c[d].reshape(BB, NP, 1), 1, 2)
            cj = jnp.broadcast_to(cl, (BB, NP, NP)).reshape(R, NP)
            dd = jnp.broadcast_to(c[d], (R, NP)) - cj
            diffs.append(dd)
            radial = dd * dd if radial is None else radial + dd * dd
        return diffs, radial

    d0, ea_j = pair_geom(crd)

    for l in range(N_LAYERS):
        if l == 0:
            diffs, radial_j = d0, ea_j
        else:
            diffs, radial_j = pair_geom(crd)
        inv = 1.0 / (jnp.sqrt(radial_j + 1e-8) + 1.0)

        hb = h.astype(bf16)
        hwa_t = jnp.dot(hb, wat_ref[l], preferred_element_type=f32)
        if l == 0:
            hwb_pk = jnp.dot(h0.astype(bf16), wbt_ref[...],
                             preferred_element_type=f32) + b1t_ref[l]
        else:
            hwb = jnp.dot(hb, wb_ref[l], preferred_element_type=f32)
            hwb3 = hwb.reshape(BB, NP, HID)
            hwb_pk = jnp.concatenate([hwb3[:, j, :] for j in range(NP)],
                                     axis=1) + b1t_ref[l]
        hwb_b = jnp.broadcast_to(hwb_pk[:, None, :],
                                 (BB, NP, LW)).reshape(R, LW)
        if l == 0:
            spread = jnp.dot(radial_j.astype(bf16), wre0_ref[...],
                             preferred_element_type=f32)
        else:
            spread = (jnp.dot(radial_j.astype(bf16), wr_ref[l],
                              preferred_element_type=f32)
                      + jnp.dot(ea_j.astype(bf16), we_ref[l],
                                preferred_element_type=f32))
        e_lin = hwa_t + hwb_b + spread
        m1 = _silu(e_lin.astype(bf16))
        m = _silu(jnp.dot(m1, bdw2_ref[l],
                          preferred_element_type=f32).astype(bf16)
                  + b2t_ref[l].astype(bf16))
        s1 = _silu(jnp.dot(m, bdc1_ref[l],
                           preferred_element_type=f32).astype(bf16)
                   + c1bt_ref[l].astype(bf16))
        scal_j = jnp.dot(s1, c2s_ref[l],
                         preferred_element_type=f32)
        w = inv * jnp.where(mask_j, scal_j, 0.0)
        for d in range(DIM):
            upd = jnp.sum(diffs[d] * w, axis=1, keepdims=True)
            crd[d] = crd[d] + upd

        if l < N_LAYERS - 1:
            m_masked = jnp.where(mask_w, m, jnp.bfloat16(0.0))
            agg = jnp.dot(m_masked, summ_ref[...],
                          preferred_element_type=f32)
            n1 = _silu(jnp.dot(hb, wn1h_ref[l], preferred_element_type=f32)
                       + jnp.dot(agg.astype(bf16), wn1a_ref[l],
                                 preferred_element_type=f32)
                       + bn1_ref[l])
            h = h + jnp.dot(n1.astype(bf16), wn2_ref[l],
                            preferred_element_type=f32) + bn2_ref[l]

    nmask = jax.lax.broadcasted_iota(jnp.int32, (1, NP, 1), 1) < N_PART
    xp = jnp.concatenate(crd, axis=1).reshape(BB, NP, DIM)
    vec = xp - x
    vec = vec - jnp.sum(jnp.where(nmask, vec, 0.0), axis=1,
                        keepdims=True) * (1.0 / N_PART)
    c_skip = (SIGMA_DATA ** 2) * (c_in * c_in)
    c_out = t * SIGMA_DATA * c_in
    x0 = c_skip[:, :, None] * xt + c_out[:, :, None] * vec
    x0 = x0 - jnp.sum(jnp.where(nmask, x0, 0.0), axis=1,
                      keepdims=True) * (1.0 / N_PART)
    out_ref[...] = x0


@jax.jit
def kernel(xt, t, params):
    B = xt.shape[0]
    xt_p = jnp.pad(xt, ((0, 0), (0, NP - N_PART), (0, 0)))
    t2 = t[:, None]

    L = params["layers"]
    eye = jnp.eye(NP, dtype=jnp.float32)
    stk = lambda f: jnp.stack([f(lp) for lp in L])
    bcast = jnp.bfloat16
    wat = stk(lambda lp: jnp.tile(lp["edge_w1"][:HID],
                                  (1, NP))).astype(bcast)
    wb = stk(lambda lp: lp["edge_w1"][HID:2 * HID]).astype(bcast)
    wr = stk(lambda lp: jnp.kron(eye, lp["edge_w1"][2 * HID:2 * HID + 1])
             ).astype(bcast)
    we = stk(lambda lp: jnp.kron(eye, lp["edge_w1"][2 * HID + 1:2 * HID + 2])
             ).astype(bcast)
    wbt = jnp.tile(L[0]["edge_w1"][HID:2 * HID], (1, NP)).astype(bcast)
    wre0 = jnp.kron(eye, L[0]["edge_w1"][2 * HID:2 * HID + 1]
                    + L[0]["edge_w1"][2 * HID + 1:2 * HID + 2]).astype(bcast)
    b1t = stk(lambda lp: jnp.tile(lp["edge_b1"][None], (1, NP)))
    bdw2 = stk(lambda lp: jnp.kron(eye, lp["edge_w2"])).astype(jnp.bfloat16)
    b2t = stk(lambda lp: jnp.tile(lp["edge_b2"][None], (1, NP)))
    bdc1 = stk(lambda lp: jnp.kron(eye, lp["coord_w1"])).astype(jnp.bfloat16)
    c1bt = stk(lambda lp: jnp.tile(lp["coord_b1"][None], (1, NP)))
    c2s = stk(lambda lp: jnp.kron(eye, lp["coord_w2"])).astype(jnp.bfloat16)
    summ = jnp.tile(jnp.eye(HID, dtype=jnp.float32),
                    (NP, 1)).astype(jnp.bfloat16)
    wn1h = stk(lambda lp: lp["node_w1"][:HID]).astype(bcast)
    wn1a = stk(lambda lp: lp["node_w1"][HID:]).astype(bcast)
    bn1 = stk(lambda lp: lp["node_b1"][None])
    wn2 = stk(lambda lp: lp["node_w2"]).astype(bcast)
    bn2 = stk(lambda lp: lp["node_b2"][None])
    wsin = params["emb_w"][:HID]
    wcos = params["emb_w"][HID:]
    embb = params["emb_b"][None]

    grid = B // BB
    full = lambda s: pl.BlockSpec(s, lambda b: (0,) * len(s))
    out = pl.pallas_call(
        _fused_kernel,
        grid=(grid,),
        in_specs=[
            pl.BlockSpec((BB, NP, DIM), lambda b: (b, 0, 0)),
            pl.BlockSpec((BB, 1), lambda b: (b, 0)),
            full((HID, HID)), full((HID, HID)), full((1, HID)),
            full((N_LAYERS, HID, LW)), full((N_LAYERS, HID, HID)),
            full((HID, LW)), full((NP, LW)),
            full((N_LAYERS, NP, LW)), full((N_LAYERS, NP, LW)),
            full((N_LAYERS, 1, LW)),
            full((N_LAYERS, LW, LW)), full((N_LAYERS, 1, LW)),
            full((N_LAYERS, LW, LW)), full((N_LAYERS, 1, LW)),
            full((N_LAYERS, LW, NP)), full((LW, HID)),
            full((N_LAYERS, HID, HID)), full((N_LAYERS, HID, HID)),
            full((N_LAYERS, 1, HID)),
            full((N_LAYERS, HID, HID)), full((N_LAYERS, 1, HID)),
        ],
        out_specs=pl.BlockSpec((BB, NP, DIM), lambda b: (b, 0, 0)),
        out_shape=jax.ShapeDtypeStruct((B, NP, DIM), jnp.float32),
    )(xt_p, t2, wsin, wcos, embb, wat, wb, wbt, wre0, wr, we, b1t,
      bdw2, b2t, bdc1, c1bt, c2s, summ, wn1h, wn1a, bn1, wn2, bn2)
    return out[:, :N_PART, :]
